# baseline (speedup 1.0000x reference)
import functools

import jax
import jax.numpy as jnp
from jax import lax
from jax.experimental import pallas as pl
from jax.experimental.pallas import tpu as pltpu
from jax.experimental.pallas import tpu_sc as plsc

_TEMP_SCALE = 10.0
_TAU = 0.8
_EPS = 1e-20


def _med3(a, b, c):
    return jnp.maximum(jnp.minimum(a, b), jnp.minimum(jnp.maximum(a, b), c))


def _med5(a, b, c, d, e):
    f = jnp.maximum(jnp.minimum(a, b), jnp.minimum(c, d))
    g = jnp.minimum(jnp.maximum(a, b), jnp.maximum(c, d))
    return _med3(e, f, g)


def _sample_body(x_ref, u_ref, w_ref, b_ref, post_ref, ind_ref):
    h = jnp.dot(w_ref[...], x_ref[...], preferred_element_type=jnp.float32)
    h = h + b_ref[...]
    z = h / _TEMP_SCALE
    m = jnp.max(z, axis=0, keepdims=True)
    p = jnp.exp(z - m)
    p = p / jnp.sum(p, axis=0, keepdims=True)
    post_ref[...] = p
    logits = jnp.log(p)
    g = -jnp.log(-jnp.log(u_ref[...] + _EPS) + _EPS)
    zz = (logits + g) / _TAU
    mm = jnp.max(zz, axis=0, keepdims=True)
    yy = jnp.exp(zz - mm)
    yy = yy / jnp.sum(yy, axis=0, keepdims=True)
    ind_ref[...] = (yy[1:2, :] > yy[0:1, :]).astype(jnp.float32)


def _make_sc_pool(B, C, T, L, NW):
    n_cgrp = C // L
    n_grp = B * n_cgrp
    grp_per_w = n_grp // NW
    mesh = plsc.VectorSubcoreMesh(core_axis_name="c", subcore_axis_name="s")

    @functools.partial(
        pl.kernel, mesh=mesh,
        compiler_params=pltpu.CompilerParams(needs_layout_passes=False),
        out_type=jax.ShapeDtypeStruct((B * C * T,), jnp.float32),
        scratch_types=[
            pltpu.VMEM((L * T,), jnp.float32),
            pltpu.VMEM(((T + 4) * L,), jnp.float32),
            pltpu.VMEM(((T + 4) * L,), jnp.float32),
            pltpu.VMEM((T,), jnp.float32),
        ],
    )
    def sc_pool(ind_hbm, e_hbm, out_hbm, ebuf, tbuf, pbuf, indbuf):
        wid = lax.axis_index("s") * 2 + lax.axis_index("c")
        lanes = jnp.arange(L, dtype=jnp.int32)

        for gi in range(grp_per_w):
            g = wid * grp_per_w + gi
            b = g // n_cgrp
            base = (b * C + (g % n_cgrp) * L) * T
            pltpu.sync_copy(e_hbm.at[pl.ds(base, L * T)], ebuf)
            pltpu.sync_copy(ind_hbm.at[pl.ds(b * T, T)], indbuf)

            def build(t, _):
                tt = jnp.full((L,), t, jnp.int32)
                col = plsc.load_gather(ebuf, [lanes * T + tt])
                iv = plsc.load_gather(indbuf, [tt])
                tbuf[pl.ds((t + 2) * L, L)] = col * iv
                return 0

            lax.fori_loop(0, T, build, 0)

            def reflect(buf):
                buf[pl.ds(0, L)] = buf[pl.ds(4 * L, L)]
                buf[pl.ds(L, L)] = buf[pl.ds(3 * L, L)]
                buf[pl.ds((T + 2) * L, L)] = buf[pl.ds(T * L, L)]
                buf[pl.ds((T + 3) * L, L)] = buf[pl.ds((T - 1) * L, L)]

            def pool_pass(src, dst):
                reflect(src)

                def step(t, _):
                    o = t * L
                    v = _med5(src[pl.ds(o, L)], src[pl.ds(o + L, L)],
                              src[pl.ds(o + 2 * L, L)], src[pl.ds(o + 3 * L, L)],
                              src[pl.ds(o + 4 * L, L)])
                    dst[pl.ds(o + 2 * L, L)] = v
                    return 0

                lax.fori_loop(0, T, step, 0)

            pool_pass(tbuf, pbuf)
            pool_pass(pbuf, tbuf)
            reflect(tbuf)

            def last(t, _):
                o = t * L
                v = _med5(tbuf[pl.ds(o, L)], tbuf[pl.ds(o + L, L)],
                          tbuf[pl.ds(o + 2 * L, L)], tbuf[pl.ds(o + 3 * L, L)],
                          tbuf[pl.ds(o + 4 * L, L)])
                tt = jnp.full((L,), t, jnp.int32)
                plsc.store_scatter(ebuf, [lanes * T + tt], v)
                return 0

            lax.fori_loop(0, T, last, 0)
            pltpu.sync_copy(ebuf, out_hbm.at[pl.ds(base, L * T)])

    return sc_pool


@jax.jit
def kernel(x, e, u, W, b):
    B, C, T = x.shape
    ut = jnp.transpose(u, (0, 2, 1))
    b2 = jnp.reshape(b, (2, 1))

    post_t, ind = pl.pallas_call(
        _sample_body,
        grid=(B,),
        in_specs=[
            pl.BlockSpec((None, C, T), lambda i: (i, 0, 0)),
            pl.BlockSpec((None, 2, T), lambda i: (i, 0, 0)),
            pl.BlockSpec((2, C), lambda i: (0, 0)),
            pl.BlockSpec((2, 1), lambda i: (0, 0)),
        ],
        out_specs=[
            pl.BlockSpec((None, 2, T), lambda i: (i, 0, 0)),
            pl.BlockSpec((None, 1, T), lambda i: (i, 0, 0)),
        ],
        out_shape=[
            jax.ShapeDtypeStruct((B, 2, T), jnp.float32),
            jax.ShapeDtypeStruct((B, 1, T), jnp.float32),
        ],
    )(x, ut, W, b2)

    info = plsc.get_sparse_core_info()
    NW = info.num_cores * info.num_subcores
    L = info.num_lanes
    mask1d = _make_sc_pool(B, C, T, L, NW)(
        jnp.reshape(ind, (B * T,)), jnp.reshape(e, (B * C * T,)))
    mask = jnp.reshape(mask1d, (B, C, T))

    posterior = jnp.transpose(post_t, (0, 2, 1))
    return posterior, mask


# speedup vs baseline: 1.0827x; 1.0827x over previous
import functools

import jax
import jax.numpy as jnp
from jax import lax
from jax.experimental import pallas as pl
from jax.experimental.pallas import tpu as pltpu
from jax.experimental.pallas import tpu_sc as plsc

_TEMP_SCALE = 10.0
_TAU = 0.8
_EPS = 1e-20


def _med3(a, b, c):
    return jnp.maximum(jnp.minimum(a, b), jnp.minimum(jnp.maximum(a, b), c))


def _med5(a, b, c, d, e):
    f = jnp.maximum(jnp.minimum(a, b), jnp.minimum(c, d))
    g = jnp.minimum(jnp.maximum(a, b), jnp.maximum(c, d))
    return _med3(e, f, g)


def _sample_body(x_ref, u_ref, w_ref, b_ref, post_ref, ind_ref):
    h = jnp.dot(w_ref[...], x_ref[...], preferred_element_type=jnp.float32)
    h = h + b_ref[...]
    z = h / _TEMP_SCALE
    m = jnp.max(z, axis=0, keepdims=True)
    p = jnp.exp(z - m)
    p = p / jnp.sum(p, axis=0, keepdims=True)
    post_ref[...] = p
    logits = jnp.log(p)
    g = -jnp.log(-jnp.log(u_ref[...] + _EPS) + _EPS)
    zz = (logits + g) / _TAU
    mm = jnp.max(zz, axis=0, keepdims=True)
    yy = jnp.exp(zz - mm)
    yy = yy / jnp.sum(yy, axis=0, keepdims=True)
    ind_ref[...] = (yy[1:2, :] > yy[0:1, :]).astype(jnp.float32)


def _make_sc_pool(B, C, T, L, NW):
    n_cgrp = C // L
    n_grp = B * n_cgrp
    grp_per_w = n_grp // NW
    mesh = plsc.VectorSubcoreMesh(core_axis_name="c", subcore_axis_name="s")

    @functools.partial(
        pl.kernel, mesh=mesh,
        compiler_params=pltpu.CompilerParams(needs_layout_passes=False),
        out_type=jax.ShapeDtypeStruct((B * C * T,), jnp.float32),
        scratch_types=[
            pltpu.VMEM((L * T,), jnp.float32),
            pltpu.VMEM(((T + 4) * L,), jnp.float32),
            pltpu.VMEM(((T + 4) * L,), jnp.float32),
            pltpu.VMEM((T,), jnp.float32),
        ],
    )
    def sc_pool(ind_hbm, e_hbm, out_hbm, ebuf, tbuf, pbuf, indbuf):
        wid = lax.axis_index("s") * 2 + lax.axis_index("c")
        lanes = jnp.arange(L, dtype=jnp.int32)

        for gi in range(grp_per_w):
            g = wid * grp_per_w + gi
            b = g // n_cgrp
            base = (b * C + (g % n_cgrp) * L) * T
            pltpu.sync_copy(e_hbm.at[pl.ds(base, L * T)], ebuf)
            pltpu.sync_copy(ind_hbm.at[pl.ds(b * T, T)], indbuf)

            U = 8
            lanesT = lanes * T

            def build(tu, _):
                t0 = tu * U
                for k in range(U):
                    t = t0 + k
                    tt = jnp.full((L,), t, jnp.int32)
                    col = plsc.load_gather(ebuf, [lanesT + tt])
                    iv = plsc.load_gather(indbuf, [tt])
                    tbuf[pl.ds((t + 2) * L, L)] = col * iv
                return 0

            lax.fori_loop(0, T // U, build, 0)

            def reflect(buf):
                buf[pl.ds(0, L)] = buf[pl.ds(4 * L, L)]
                buf[pl.ds(L, L)] = buf[pl.ds(3 * L, L)]
                buf[pl.ds((T + 2) * L, L)] = buf[pl.ds(T * L, L)]
                buf[pl.ds((T + 3) * L, L)] = buf[pl.ds((T - 1) * L, L)]

            def pool_pass(src, dst):
                reflect(src)

                def step(tu, _):
                    for k in range(U):
                        o = (tu * U + k) * L
                        v = _med5(src[pl.ds(o, L)], src[pl.ds(o + L, L)],
                                  src[pl.ds(o + 2 * L, L)],
                                  src[pl.ds(o + 3 * L, L)],
                                  src[pl.ds(o + 4 * L, L)])
                        dst[pl.ds(o + 2 * L, L)] = v
                    return 0

                lax.fori_loop(0, T // U, step, 0)

            pool_pass(tbuf, pbuf)
            pool_pass(pbuf, tbuf)
            reflect(tbuf)

            def last(tu, _):
                for k in range(U):
                    t = tu * U + k
                    o = t * L
                    v = _med5(tbuf[pl.ds(o, L)], tbuf[pl.ds(o + L, L)],
                              tbuf[pl.ds(o + 2 * L, L)],
                              tbuf[pl.ds(o + 3 * L, L)],
                              tbuf[pl.ds(o + 4 * L, L)])
                    tt = jnp.full((L,), t, jnp.int32)
                    plsc.store_scatter(ebuf, [lanesT + tt], v)
                return 0

            lax.fori_loop(0, T // U, last, 0)
            pltpu.sync_copy(ebuf, out_hbm.at[pl.ds(base, L * T)])

    return sc_pool


@jax.jit
def kernel(x, e, u, W, b):
    B, C, T = x.shape
    ut = jnp.transpose(u, (0, 2, 1))
    b2 = jnp.reshape(b, (2, 1))

    post_t, ind = pl.pallas_call(
        _sample_body,
        grid=(B,),
        in_specs=[
            pl.BlockSpec((None, C, T), lambda i: (i, 0, 0)),
            pl.BlockSpec((None, 2, T), lambda i: (i, 0, 0)),
            pl.BlockSpec((2, C), lambda i: (0, 0)),
            pl.BlockSpec((2, 1), lambda i: (0, 0)),
        ],
        out_specs=[
            pl.BlockSpec((None, 2, T), lambda i: (i, 0, 0)),
            pl.BlockSpec((None, 1, T), lambda i: (i, 0, 0)),
        ],
        out_shape=[
            jax.ShapeDtypeStruct((B, 2, T), jnp.float32),
            jax.ShapeDtypeStruct((B, 1, T), jnp.float32),
        ],
    )(x, ut, W, b2)

    info = plsc.get_sparse_core_info()
    NW = info.num_cores * info.num_subcores
    L = info.num_lanes
    mask1d = _make_sc_pool(B, C, T, L, NW)(
        jnp.reshape(ind, (B * T,)), jnp.reshape(e, (B * C * T,)))
    mask = jnp.reshape(mask1d, (B, C, T))

    posterior = jnp.transpose(post_t, (0, 2, 1))
    return posterior, mask


# speedup vs baseline: 9.9292x; 9.1708x over previous
import jax
import jax.numpy as jnp
from jax.experimental import pallas as pl
from jax.experimental.pallas import tpu as pltpu

_TEMP_SCALE = 10.0
_TAU = 0.8
_EPS = 1e-20


def _med3(a, b, c):
    return jnp.maximum(jnp.minimum(a, b), jnp.minimum(jnp.maximum(a, b), c))


def _med5(a, b, c, d, e):
    f = jnp.maximum(jnp.minimum(a, b), jnp.minimum(c, d))
    g = jnp.minimum(jnp.maximum(a, b), jnp.maximum(c, d))
    return _med3(e, f, g)


def _sample_body(x_ref, u_ref, w_ref, b_ref, post_ref, ind_ref):
    h = jnp.dot(w_ref[...], x_ref[...], preferred_element_type=jnp.float32)
    h = h + b_ref[...]
    z = h / _TEMP_SCALE
    m = jnp.max(z, axis=0, keepdims=True)
    p = jnp.exp(z - m)
    p = p / jnp.sum(p, axis=0, keepdims=True)
    post_ref[...] = p
    logits = jnp.log(p)
    g = -jnp.log(-jnp.log(u_ref[...] + _EPS) + _EPS)
    zz = (logits + g) / _TAU
    mm = jnp.max(zz, axis=0, keepdims=True)
    yy = jnp.exp(zz - mm)
    yy = yy / jnp.sum(yy, axis=0, keepdims=True)
    ind_ref[...] = (yy[1:2, :] > yy[0:1, :]).astype(jnp.float32)


def _pool_body(ind_ref, e_ref, out_ref, pad_ref):
    T = e_ref.shape[1]
    masked = ind_ref[...] * e_ref[...]
    x = masked.T
    for _ in range(3):
        pad_ref[2:T + 2, :] = x
        pad_ref[0:1, :] = pad_ref[4:5, :]
        pad_ref[1:2, :] = pad_ref[3:4, :]
        pad_ref[T + 2:T + 3, :] = pad_ref[T:T + 1, :]
        pad_ref[T + 3:T + 4, :] = pad_ref[T - 1:T, :]
        x = _med5(
            pad_ref[0:T, :], pad_ref[1:T + 1, :], pad_ref[2:T + 2, :],
            pad_ref[3:T + 3, :], pad_ref[4:T + 4, :],
        )
    out_ref[...] = x.T


@jax.jit
def kernel(x, e, u, W, b):
    B, C, T = x.shape
    ut = jnp.transpose(u, (0, 2, 1))
    b2 = jnp.reshape(b, (2, 1))

    post_t, ind = pl.pallas_call(
        _sample_body,
        grid=(B,),
        in_specs=[
            pl.BlockSpec((None, C, T), lambda i: (i, 0, 0)),
            pl.BlockSpec((None, 2, T), lambda i: (i, 0, 0)),
            pl.BlockSpec((2, C), lambda i: (0, 0)),
            pl.BlockSpec((2, 1), lambda i: (0, 0)),
        ],
        out_specs=[
            pl.BlockSpec((None, 2, T), lambda i: (i, 0, 0)),
            pl.BlockSpec((None, 1, T), lambda i: (i, 0, 0)),
        ],
        out_shape=[
            jax.ShapeDtypeStruct((B, 2, T), jnp.float32),
            jax.ShapeDtypeStruct((B, 1, T), jnp.float32),
        ],
    )(x, ut, W, b2)

    CB = 128
    mask = pl.pallas_call(
        _pool_body,
        grid=(B, C // CB),
        in_specs=[
            pl.BlockSpec((None, 1, T), lambda i, j: (i, 0, 0)),
            pl.BlockSpec((None, CB, T), lambda i, j: (i, j, 0)),
        ],
        out_specs=pl.BlockSpec((None, CB, T), lambda i, j: (i, j, 0)),
        out_shape=jax.ShapeDtypeStruct((B, C, T), jnp.float32),
        scratch_shapes=[pltpu.VMEM((T + 8, CB), jnp.float32)],
    )(ind, e)

    posterior = jnp.transpose(post_t, (0, 2, 1))
    return posterior, mask


# speedup vs baseline: 9.9466x; 1.0017x over previous
import jax
import jax.numpy as jnp
from jax.experimental import pallas as pl
from jax.experimental.pallas import tpu as pltpu

_TEMP_SCALE = 10.0
_TAU = 0.8
_EPS = 1e-20


def _med3(a, b, c):
    return jnp.maximum(jnp.minimum(a, b), jnp.minimum(jnp.maximum(a, b), c))


def _med5(a, b, c, d, e):
    f = jnp.maximum(jnp.minimum(a, b), jnp.minimum(c, d))
    g = jnp.minimum(jnp.maximum(a, b), jnp.maximum(c, d))
    return _med3(e, f, g)


def _sample_body(x_ref, u_ref, w_ref, b_ref, post_ref, ind_ref):
    h = jnp.dot(w_ref[...], x_ref[...], preferred_element_type=jnp.float32)
    h = h + b_ref[...]
    z = h / _TEMP_SCALE
    m = jnp.max(z, axis=0, keepdims=True)
    p = jnp.exp(z - m)
    p = p / jnp.sum(p, axis=0, keepdims=True)
    post_ref[...] = p
    logits = jnp.log(p)
    g = -jnp.log(-jnp.log(u_ref[...] + _EPS) + _EPS)
    zz = (logits + g) / _TAU
    mm = jnp.max(zz, axis=0, keepdims=True)
    yy = jnp.exp(zz - mm)
    yy = yy / jnp.sum(yy, axis=0, keepdims=True)
    ind_ref[...] = (yy[1:2, :] > yy[0:1, :]).astype(jnp.float32)


def _pool_body(ind_ref, e_ref, out_ref, pad_a, pad_b):
    T = e_ref.shape[1]
    masked = ind_ref[...] * e_ref[...]
    x = masked.T
    for pad_ref in (pad_a, pad_b, pad_a):
        pad_ref[2:T + 2, :] = x
        pad_ref[0:1, :] = pad_ref[4:5, :]
        pad_ref[1:2, :] = pad_ref[3:4, :]
        pad_ref[T + 2:T + 3, :] = pad_ref[T:T + 1, :]
        pad_ref[T + 3:T + 4, :] = pad_ref[T - 1:T, :]
        x = _med5(
            pad_ref[0:T, :], pad_ref[1:T + 1, :], pad_ref[2:T + 2, :],
            pad_ref[3:T + 3, :], pad_ref[4:T + 4, :],
        )
    out_ref[...] = x.T


@jax.jit
def kernel(x, e, u, W, b):
    B, C, T = x.shape
    ut = jnp.transpose(u, (0, 2, 1))
    b2 = jnp.reshape(b, (2, 1))

    post_t, ind = pl.pallas_call(
        _sample_body,
        grid=(B,),
        in_specs=[
            pl.BlockSpec((None, C, T), lambda i: (i, 0, 0)),
            pl.BlockSpec((None, 2, T), lambda i: (i, 0, 0)),
            pl.BlockSpec((2, C), lambda i: (0, 0)),
            pl.BlockSpec((2, 1), lambda i: (0, 0)),
        ],
        out_specs=[
            pl.BlockSpec((None, 2, T), lambda i: (i, 0, 0)),
            pl.BlockSpec((None, 1, T), lambda i: (i, 0, 0)),
        ],
        out_shape=[
            jax.ShapeDtypeStruct((B, 2, T), jnp.float32),
            jax.ShapeDtypeStruct((B, 1, T), jnp.float32),
        ],
    )(x, ut, W, b2)

    CB = 128
    mask = pl.pallas_call(
        _pool_body,
        grid=(B, C // CB),
        in_specs=[
            pl.BlockSpec((None, 1, T), lambda i, j: (i, 0, 0)),
            pl.BlockSpec((None, CB, T), lambda i, j: (i, j, 0)),
        ],
        out_specs=pl.BlockSpec((None, CB, T), lambda i, j: (i, j, 0)),
        out_shape=jax.ShapeDtypeStruct((B, C, T), jnp.float32),
        scratch_shapes=[pltpu.VMEM((T + 8, CB), jnp.float32),
                        pltpu.VMEM((T + 8, CB), jnp.float32)],
    )(ind, e)

    posterior = jnp.transpose(post_t, (0, 2, 1))
    return posterior, mask
